# baseline (speedup 1.0000x reference)
import functools
import math

import jax
import jax.numpy as jnp
from jax import lax
from jax.experimental import pallas as pl
from jax.experimental.pallas import tpu as pltpu
from jax.experimental.pallas import tpu_sc as plsc

EMB = 64
HALF = 32
BATCH = 16384
MAX_PERIOD = 10000.0
TMAX = 1024

_NC = 2
_NS = 16
_NW = _NC * _NS
B_TC = 4096
B_SC = BATCH - B_TC
_BPW = B_SC // _NW
_RING = 8
_SEG = 128
_NSEG = _BPW // _SEG
_NGRP = _SEG // _RING


def _emb_body(o_ref):
    r = lax.broadcasted_iota(jnp.int32, (HALF, TMAX), 0)
    t = lax.broadcasted_iota(jnp.int32, (HALF, TMAX), 1).astype(jnp.float32)
    k0 = jnp.where(r < 16, 2 * r, 2 * r - HALF).astype(jnp.float32)
    c = -math.log(MAX_PERIOD) / HALF
    a0 = t * jnp.exp(k0 * c)
    a1 = t * jnp.exp((k0 + 1.0) * c)
    v0 = jnp.where(r < 16, jnp.cos(a0), jnp.sin(a0))
    v1 = jnp.where(r < 16, jnp.cos(a1), jnp.sin(a1))
    u0 = lax.bitcast_convert_type(v0.astype(jnp.bfloat16), jnp.uint16)
    u1 = lax.bitcast_convert_type(v1.astype(jnp.bfloat16), jnp.uint16)
    packed = u0.astype(jnp.int32) | lax.shift_left(u1.astype(jnp.int32), 16)
    o_ref[:, :] = packed


_emb_table = pl.pallas_call(
    _emb_body,
    out_shape=jax.ShapeDtypeStruct((HALF, TMAX), jnp.int32),
)


@functools.cache
def _make_sc_gather():
    mesh = plsc.VectorSubcoreMesh(core_axis_name="c", subcore_axis_name="s")

    slab_types = [pltpu.VMEM((EMB, 128), jnp.float32) for _ in range(_RING)]
    gsem_types = [pltpu.SemaphoreType.DMA for _ in range(_RING)]

    @functools.partial(
        pl.kernel,
        mesh=mesh,
        out_type=jax.ShapeDtypeStruct((EMB, B_SC), jnp.float32),
        scratch_types=[
            pltpu.VMEM((_BPW + 16, ), jnp.int32),
            pltpu.VMEM((_BPW + 16, ), jnp.int32),
            pltpu.VMEM((HALF, TMAX), jnp.int32),
            pltpu.VMEM((EMB, 128), jnp.float32),
            pltpu.VMEM((EMB, 128), jnp.float32),
            *slab_types,
            *gsem_types,
            pltpu.SemaphoreType.DMA,
            pltpu.SemaphoreType.DMA,
        ],
        compiler_params=pltpu.CompilerParams(needs_layout_passes=False),
    )
    def _sc_gather(labels_hbm, times_hbm, embp_hbm, tablet_hbm, out_hbm,
                   labels_v, times_v, emb_v, col_a, col_b, *rest):
        slabs = rest[:_RING]
        gsems = rest[_RING:2 * _RING]
        osems = rest[2 * _RING:]
        cols = (col_a, col_b)
        wid = lax.axis_index("s") * _NC + lax.axis_index("c")
        base = wid * _BPW
        pltpu.sync_copy(labels_hbm.at[pl.ds(base, _BPW)],
                        labels_v.at[pl.ds(0, _BPW)])
        pltpu.sync_copy(times_hbm.at[pl.ds(base, _BPW)],
                        times_v.at[pl.ds(0, _BPW)])
        pltpu.sync_copy(embp_hbm, emb_v)

        rows16 = [lax.iota(jnp.int32, 16) + 16 * q for q in range(4)]
        rhalf = [lax.shift_right_logical(rows16[q], 1) for q in range(4)]
        odd16 = [(rows16[q] & 1) == 1 for q in range(4)]
        himask = jnp.full((16,), -65536, jnp.int32)

        def fire(slot, lbl):
            off = pl.multiple_of(
                lax.shift_right_logical(lbl, 7) * 128, 128)
            pltpu.async_copy(
                tablet_hbm.at[:, pl.ds(off, 128)], slabs[slot], gsems[slot])

        def wait_slot(slot):
            pltpu.make_async_copy(
                tablet_hbm.at[:, pl.ds(0, 128)], slabs[slot],
                gsems[slot]).wait()

        def extract(slot, lbl, tstep, colbuf, col):
            lane = jnp.broadcast_to(lbl & 127, (16,))
            tlane = jnp.broadcast_to(tstep, (16,))
            colv = jnp.broadcast_to(col, (16,))
            for q in range(4):
                vals = plsc.load_gather(slabs[slot], [rows16[q], lane])
                w = plsc.load_gather(emb_v, [rhalf[q], tlane])
                lo = plsc.bitcast(lax.shift_left(w, 16), jnp.float32)
                hi = plsc.bitcast(w & himask, jnp.float32)
                emb = jnp.where(odd16[q], hi, lo)
                plsc.store_scatter(colbuf, [rows16[q], colv], vals + emb)

        out_copies = [None, None]
        for s in range(_NSEG):
            cb = s % 2
            if out_copies[cb] is not None:
                out_copies[cb].wait()
            lv0 = labels_v[pl.ds(s * _SEG, 16)]
            for j in range(_RING):
                fire(j, lv0[j])

            def group(c, carry, s=s, cb=cb):
                lv = labels_v[pl.ds(s * _SEG + c * _RING, 16)]
                tv = times_v[pl.ds(s * _SEG + c * _RING, 16)]
                for j in range(_RING):
                    wait_slot(j)
                    extract(j, lv[j], tv[j], cols[cb], c * _RING + j)

                    @pl.when(c < _NGRP - 1)
                    def _():
                        fire(j, lv[_RING + j])
                return carry

            lax.fori_loop(0, _NGRP, group, 0)
            out_copies[cb] = pltpu.async_copy(
                cols[cb],
                out_hbm.at[:, pl.ds(base + s * _SEG, _SEG)],
                osems[cb],
            )
        out_copies[0].wait()
        out_copies[1].wait()

    return _sc_gather


_LPS = 8
_TSTEPS = B_TC // _LPS


def _tc_gather_body(lbls_ref, *refs):
    tabs = refs[:_LPS]
    ts_ref = refs[_LPS]
    o_ref = refs[_LPS + 1]
    i = pl.program_id(0)
    t8 = ts_ref[0, 0, :].astype(jnp.float32).reshape(1, _LPS)
    j = lax.broadcasted_iota(jnp.int32, (EMB, _LPS), 0)
    kk = jnp.where(j < HALF, j, j - HALF).astype(jnp.float32)
    freqs = jnp.exp(kk * (-math.log(MAX_PERIOD) / HALF))
    args = jnp.broadcast_to(t8, (EMB, _LPS)) * freqs
    emb = jnp.where(j < HALF, jnp.cos(args), jnp.sin(args))
    lane_iota = lax.broadcasted_iota(jnp.int32, (EMB, 128), 1)
    cols = []
    for k in range(_LPS):
        lbl = lbls_ref[i * _LPS + k]
        sel = (lane_iota == (lbl & 127)).astype(jnp.float32)
        cols.append(jnp.sum(tabs[k][:, :] * sel, axis=1, keepdims=True))
    o_ref[0, :, :] = emb + jnp.concatenate(cols, axis=1)


def _make_tab_spec(k):
    return pl.BlockSpec(
        (EMB, 128),
        lambda i, lbls, k=k: (0, lax.shift_right_logical(lbls[i * _LPS + k], 7)),
    )


_tc_gather = pl.pallas_call(
    _tc_gather_body,
    grid_spec=pltpu.PrefetchScalarGridSpec(
        num_scalar_prefetch=1,
        grid=(_TSTEPS,),
        in_specs=[
            *[_make_tab_spec(k) for k in range(_LPS)],
            pl.BlockSpec((1, 1, _LPS), lambda i, lbls: (i, 0, 0)),
        ],
        out_specs=pl.BlockSpec((1, EMB, _LPS), lambda i, lbls: (i, 0, 0)),
    ),
    out_shape=jax.ShapeDtypeStruct((_TSTEPS, EMB, _LPS), jnp.float32),
)


def kernel(timesteps, labels, class_embedding):
    embp = _emb_table()
    tablet = class_embedding.T
    out_sc = _make_sc_gather()(labels, timesteps, embp, tablet)
    lbl_tc = labels[B_SC:]
    ts_tc = timesteps[B_SC:].reshape(_TSTEPS, 1, _LPS)
    out_tc3 = _tc_gather(lbl_tc, *([tablet] * _LPS), ts_tc)
    out_tc = out_tc3.transpose(0, 2, 1).reshape(B_TC, EMB)
    return jnp.concatenate([out_sc.T, out_tc], axis=0)


# speedup vs baseline: 1.7617x; 1.7617x over previous
import functools
import math

import jax
import jax.numpy as jnp
from jax import lax
from jax.experimental import pallas as pl
from jax.experimental.pallas import tpu as pltpu
from jax.experimental.pallas import tpu_sc as plsc

EMB = 64
HALF = 32
BATCH = 16384
MAX_PERIOD = 10000.0
TMAX = 1024

_NC = 2
_NS = 16
_NW = _NC * _NS
_BPW = BATCH // _NW
_RING = 8
_SEG = 128
_NSEG = _BPW // _SEG
_NGRP = _SEG // _RING


def _emb_body(o_ref):
    r = lax.broadcasted_iota(jnp.int32, (HALF, TMAX), 0)
    t = lax.broadcasted_iota(jnp.int32, (HALF, TMAX), 1).astype(jnp.float32)
    k0 = jnp.where(r < 16, 2 * r, 2 * r - HALF).astype(jnp.float32)
    c = -math.log(MAX_PERIOD) / HALF
    a0 = t * jnp.exp(k0 * c)
    a1 = t * jnp.exp((k0 + 1.0) * c)
    v0 = jnp.where(r < 16, jnp.cos(a0), jnp.sin(a0))
    v1 = jnp.where(r < 16, jnp.cos(a1), jnp.sin(a1))
    u0 = lax.bitcast_convert_type(v0.astype(jnp.bfloat16), jnp.uint16)
    u1 = lax.bitcast_convert_type(v1.astype(jnp.bfloat16), jnp.uint16)
    packed = u0.astype(jnp.int32) | lax.shift_left(u1.astype(jnp.int32), 16)
    o_ref[:, :] = packed


_emb_table = pl.pallas_call(
    _emb_body,
    out_shape=jax.ShapeDtypeStruct((HALF, TMAX), jnp.int32),
)


@functools.cache
def _make_sc_gather():
    mesh = plsc.VectorSubcoreMesh(core_axis_name="c", subcore_axis_name="s")

    slab_types = [pltpu.VMEM((EMB, 128), jnp.float32) for _ in range(_RING)]
    gsem_types = [pltpu.SemaphoreType.DMA for _ in range(_RING)]

    @functools.partial(
        pl.kernel,
        mesh=mesh,
        out_type=jax.ShapeDtypeStruct((EMB, BATCH), jnp.float32),
        scratch_types=[
            pltpu.VMEM((_BPW + 8, ), jnp.int32),
            pltpu.VMEM((_BPW + 8, ), jnp.int32),
            pltpu.VMEM((HALF, TMAX), jnp.int32),
            pltpu.VMEM((EMB, 128), jnp.float32),
            pltpu.VMEM((EMB, 128), jnp.float32),
            *slab_types,
            *gsem_types,
            pltpu.SemaphoreType.DMA,
            pltpu.SemaphoreType.DMA,
        ],
        compiler_params=pltpu.CompilerParams(needs_layout_passes=False),
    )
    def _sc_gather(labels_hbm, times_hbm, embp_hbm, tablet_hbm, out_hbm,
                   labels_v, times_v, emb_v, col_a, col_b, *rest):
        slabs = rest[:_RING]
        gsems = rest[_RING:2 * _RING]
        osems = rest[2 * _RING:]
        cols = (col_a, col_b)
        wid = lax.axis_index("s") * _NC + lax.axis_index("c")
        base = wid * _BPW
        pltpu.sync_copy(labels_hbm.at[pl.ds(base, _BPW)],
                        labels_v.at[pl.ds(0, _BPW)])
        pltpu.sync_copy(times_hbm.at[pl.ds(base, _BPW)],
                        times_v.at[pl.ds(0, _BPW)])
        pltpu.sync_copy(embp_hbm, emb_v)

        rows16 = [lax.iota(jnp.int32, 16) + 16 * q for q in range(4)]
        rhalf = [lax.shift_right_logical(rows16[q], 1) for q in range(4)]
        odd16 = [(rows16[q] & 1) == 1 for q in range(4)]
        himask = jnp.full((16,), -65536, jnp.int32)

        def fire(slot, lbl):
            off = pl.multiple_of(
                lax.shift_right_logical(lbl, 7) * 128, 128)
            pltpu.async_copy(
                tablet_hbm.at[:, pl.ds(off, 128)], slabs[slot], gsems[slot])

        def wait_slot(slot):
            pltpu.make_async_copy(
                tablet_hbm.at[:, pl.ds(0, 128)], slabs[slot],
                gsems[slot]).wait()

        def extract(slot, lbl, tstep, colbuf, col):
            lane = jnp.broadcast_to(lbl & 127, (16,))
            tlane = jnp.broadcast_to(tstep, (16,))
            colv = jnp.broadcast_to(col, (16,))
            for q in range(4):
                vals = plsc.load_gather(slabs[slot], [rows16[q], lane])
                w = plsc.load_gather(emb_v, [rhalf[q], tlane])
                lo = plsc.bitcast(lax.shift_left(w, 16), jnp.float32)
                hi = plsc.bitcast(w & himask, jnp.float32)
                emb = jnp.where(odd16[q], hi, lo)
                plsc.store_scatter(colbuf, [rows16[q], colv], vals + emb)

        out_copies = [None, None]
        for s in range(_NSEG):
            cb = s % 2
            if out_copies[cb] is not None:
                out_copies[cb].wait()
            lv0 = labels_v[pl.ds(s * _SEG, 16)]
            for j in range(_RING):
                fire(j, lv0[j])

            def group(c, carry, s=s, cb=cb):
                lv = labels_v[pl.ds(s * _SEG + c * _RING, 16)]
                tv = times_v[pl.ds(s * _SEG + c * _RING, 16)]
                for j in range(_RING):
                    wait_slot(j)
                    extract(j, lv[j], tv[j], cols[cb], c * _RING + j)

                    @pl.when(c < _NGRP - 1)
                    def _():
                        fire(j, lv[_RING + j])
                return carry

            lax.fori_loop(0, _NGRP, group, 0)
            out_copies[cb] = pltpu.async_copy(
                cols[cb],
                out_hbm.at[:, pl.ds(base + s * _SEG, _SEG)],
                osems[cb],
            )
        out_copies[0].wait()
        out_copies[1].wait()

    return _sc_gather


def kernel(timesteps, labels, class_embedding):
    embp = _emb_table()
    out_t = _make_sc_gather()(labels, timesteps, embp, class_embedding.T)
    return out_t.T
